# baseline (speedup 1.0000x reference)
import functools

import jax
import jax.numpy as jnp
from jax import lax
from jax.experimental import pallas as pl
from jax.experimental.pallas import tpu as pltpu
from jax.experimental.pallas import tpu_sc as plsc

_N = 10000
_E = 320000
_D = 128
_H = 32
_C = 10

_NC = 2
_NS = 16
_NW = _NC * _NS

_EPW = _E // _NW
_K = 80
_NCHUNK = _EPW // _K
_G = 5
_NG = _NCHUNK // _G
_RPS = _N // _NS
_N2 = 10240
_RPS2 = _N2 // _NS

_f32 = jnp.float32

_sc_mesh = plsc.VectorSubcoreMesh(core_axis_name="c", subcore_axis_name="s")
_sc_params = pltpu.CompilerParams(use_tc_tiling_on_sc=False)


@functools.partial(
    pl.kernel,
    mesh=_sc_mesh,
    out_type=jax.ShapeDtypeStruct((2 * _N2,), _f32),
    scratch_types=[
        pltpu.VMEM((_NCHUNK, _K), jnp.int32),
        pltpu.VMEM((_K,), _f32),
        pltpu.VMEM_SHARED((_N2,), _f32),
    ],
    compiler_params=_sc_params,
)
def _sc_deg(dst_hbm, zeros1_hbm, out_hbm, didx, ones_v, acc):
    c = lax.axis_index("c")
    s = lax.axis_index("s")
    w = s * _NC + c

    pltpu.sync_copy(dst_hbm.at[w], didx)
    for j in range(_K // 16):
        ones_v[pl.ds(j * 16, 16)] = jnp.ones((16,), _f32)
    pltpu.sync_copy(zeros1_hbm.at[pl.ds(s * _RPS2, _RPS2)],
                    acc.at[pl.ds(s * _RPS2, _RPS2)])
    plsc.subcore_barrier()

    def body(i, carry):
        pltpu.sync_copy(ones_v, acc.at[didx.at[i]], add=True)
        return carry

    lax.fori_loop(0, _NCHUNK, body, 0)
    plsc.subcore_barrier()
    pltpu.sync_copy(acc.at[pl.ds(s * _RPS2, _RPS2)],
                    out_hbm.at[pl.ds(c * _N2 + s * _RPS2, _RPS2)])


@functools.partial(
    pl.kernel,
    mesh=_sc_mesh,
    out_type=jax.ShapeDtypeStruct((2 * _N2, _H), _f32),
    scratch_types=[
        pltpu.VMEM((_NCHUNK, _K), jnp.int32),
        pltpu.VMEM((_NCHUNK, _K), jnp.int32),
        pltpu.VMEM((2, _G * _K, _H), _f32),
        pltpu.VMEM_SHARED((_N2, _H), _f32),
        pltpu.SemaphoreType.DMA,
        pltpu.SemaphoreType.DMA,
    ],
    compiler_params=_sc_params,
)
def _sc_scatter(g_hbm, src_hbm, dst_hbm, zeros2_hbm, out_hbm,
                sidx, didx, rows, acc, sem0, sem1):
    c = lax.axis_index("c")
    s = lax.axis_index("s")
    w = s * _NC + c

    pltpu.sync_copy(src_hbm.at[w], sidx)
    pltpu.sync_copy(dst_hbm.at[w], didx)
    pltpu.sync_copy(zeros2_hbm.at[pl.ds(s * _RPS2, _RPS2)],
                    acc.at[pl.ds(s * _RPS2, _RPS2)])
    plsc.subcore_barrier()

    def fire(grp, slot, sem):
        for j in range(_G):
            pltpu.async_copy(g_hbm.at[sidx.at[grp * _G + j]],
                             rows.at[slot, pl.ds(j * _K, _K)], sem)

    def drain_scatter(grp, slot, sem):
        pltpu.make_async_copy(zeros2_hbm.at[pl.ds(0, _G * _K)],
                              rows.at[slot], sem).wait()
        for j in range(_G):
            pltpu.sync_copy(rows.at[slot, pl.ds(j * _K, _K)],
                            acc.at[didx.at[grp * _G + j]], add=True)

    fire(0, 0, sem0)
    fire(1, 1, sem1)

    def body(k, carry):
        g0 = 2 * k
        drain_scatter(g0, 0, sem0)
        fire(g0 + 2, 0, sem0)
        drain_scatter(g0 + 1, 1, sem1)
        fire(g0 + 3, 1, sem1)
        return carry

    lax.fori_loop(0, (_NG - 3) // 2, body, 0)
    drain_scatter(_NG - 3, 0, sem0)
    fire(_NG - 1, 0, sem0)
    drain_scatter(_NG - 2, 1, sem1)
    drain_scatter(_NG - 1, 0, sem0)
    plsc.subcore_barrier()
    pltpu.sync_copy(acc.at[pl.ds(s * _RPS2, _RPS2)],
                    out_hbm.at[pl.ds(c * _N2 + s * _RPS2, _RPS2)])


def _tc_h1_body(x0_ref, w1_ref, h1_ref):
    h1_ref[...] = jnp.dot(x0_ref[...], w1_ref[...],
                          preferred_element_type=_f32)


def _tc_g1_body(h1_ref, degp_ref, g1_ref, dinv_ref):
    deg = 1.0 + degp_ref[0] + degp_ref[1]
    dinv = lax.rsqrt(deg)
    dinv32 = jnp.broadcast_to(dinv, (_N, _H))
    g1_ref[...] = h1_ref[...] * dinv32
    dinv_ref[...] = dinv32


def _tc_mid_body(p_ref, g_ref, dinv_ref, b_ref, w_ref, gn_ref):
    dinv = dinv_ref[...]
    x = jnp.maximum(dinv * (p_ref[0] + p_ref[1] + g_ref[...]) + b_ref[...], 0.0)
    gn_ref[...] = jnp.dot(x, w_ref[...], preferred_element_type=_f32) * dinv


def _tc_head_body(p_ref, g_ref, dinv_ref, b3_ref, wl1_ref, bl1_ref,
                  wl2_ref, bl2_ref, out_ref):
    dinv = dinv_ref[...]
    x3 = jnp.maximum(dinv * (p_ref[0] + p_ref[1] + g_ref[...]) + b3_ref[...], 0.0)
    h = jnp.maximum(
        jnp.dot(x3, wl1_ref[...], preferred_element_type=_f32) + bl1_ref[...], 0.0)
    out_ref[...] = (
        jnp.dot(h, wl2_ref[...], preferred_element_type=_f32) + bl2_ref[...])


def kernel(x0, edge_index, batch, W1, b1, W2, b2, W3, b3, Wl1, bl1, Wl2, bl2):
    src = edge_index[0].reshape(_NW, _NCHUNK, _K)
    dst = edge_index[1].reshape(_NW, _NCHUNK, _K)
    zeros1 = jnp.zeros((_N2,), _f32)
    zeros2 = jnp.zeros((_N2, _H), _f32)

    h1 = pl.pallas_call(
        _tc_h1_body,
        out_shape=jax.ShapeDtypeStruct((_N, _H), _f32),
    )(x0, W1)

    degp = _sc_deg(dst, zeros1)
    degp3 = degp.reshape(2, _N2)[:, :_N].reshape(2, _N, 1)

    g1, dinv32 = pl.pallas_call(
        _tc_g1_body,
        out_shape=[jax.ShapeDtypeStruct((_N, _H), _f32),
                   jax.ShapeDtypeStruct((_N, _H), _f32)],
    )(h1, degp3)

    p1 = _sc_scatter(g1, src, dst, zeros2).reshape(2, _N2, _H)[:, :_N, :]
    g2 = pl.pallas_call(
        _tc_mid_body,
        out_shape=jax.ShapeDtypeStruct((_N, _H), _f32),
    )(p1, g1, dinv32, b1.reshape(1, _H), W2)

    p2 = _sc_scatter(g2, src, dst, zeros2).reshape(2, _N2, _H)[:, :_N, :]
    g3 = pl.pallas_call(
        _tc_mid_body,
        out_shape=jax.ShapeDtypeStruct((_N, _H), _f32),
    )(p2, g2, dinv32, b2.reshape(1, _H), W3)

    p3 = _sc_scatter(g3, src, dst, zeros2).reshape(2, _N2, _H)[:, :_N, :]
    out = pl.pallas_call(
        _tc_head_body,
        out_shape=jax.ShapeDtypeStruct((_N, _C), _f32),
    )(p3, g3, dinv32, b3.reshape(1, _H), Wl1, bl1.reshape(1, 16),
      Wl2, bl2.reshape(1, _C))
    return out


# speedup vs baseline: 1.4721x; 1.4721x over previous
import functools

import jax
import jax.numpy as jnp
from jax import lax
from jax.experimental import pallas as pl
from jax.experimental.pallas import tpu as pltpu
from jax.experimental.pallas import tpu_sc as plsc

_N = 10000
_E = 320000
_D = 128
_H = 32
_C = 10

_NC = 2
_NS = 16
_NW = _NC * _NS

_EPW = _E // _NW
_K = 80
_NCHUNK = _EPW // _K
_G = 5
_NG = _NCHUNK // _G
_N2 = 10240
_RPS2 = _N2 // _NS
_NP = _N2 // 4

_f32 = jnp.float32

_sc_mesh = plsc.VectorSubcoreMesh(core_axis_name="c", subcore_axis_name="s")
_sc_params = pltpu.CompilerParams(use_tc_tiling_on_sc=False)


@functools.partial(
    pl.kernel,
    mesh=_sc_mesh,
    out_type=jax.ShapeDtypeStruct((2 * _N2, _H), _f32),
    scratch_types=[
        pltpu.VMEM((_NCHUNK, _K), jnp.int32),
        pltpu.VMEM((_K,), _f32),
        pltpu.VMEM((_RPS2,), _f32),
        pltpu.VMEM((_RPS2, _H), _f32),
        pltpu.VMEM_SHARED((_N2,), _f32),
    ],
    compiler_params=_sc_params,
)
def _sc_deg(dst_hbm, zeros1_hbm, out_hbm, didx, ones_v, degv, dexp, acc):
    c = lax.axis_index("c")
    s = lax.axis_index("s")
    w = s * _NC + c

    pltpu.sync_copy(dst_hbm.at[w], didx)
    for j in range(_K // 16):
        ones_v[pl.ds(j * 16, 16)] = jnp.ones((16,), _f32)
    pltpu.sync_copy(zeros1_hbm.at[pl.ds(s * _RPS2, _RPS2)],
                    acc.at[pl.ds(s * _RPS2, _RPS2)])
    plsc.subcore_barrier()

    def body(i, carry):
        pltpu.sync_copy(ones_v, acc.at[didx.at[i]], add=True)
        return carry

    lax.fori_loop(0, _NCHUNK, body, 0)
    plsc.subcore_barrier()

    pltpu.sync_copy(acc.at[pl.ds(s * _RPS2, _RPS2)], degv)

    def expand(blk, carry):
        v16 = degv[pl.ds(blk * 16, 16)]
        for i in range(16):
            row = jnp.broadcast_to(v16[i], (16,))
            dexp[blk * 16 + i, pl.ds(0, 16)] = row
            dexp[blk * 16 + i, pl.ds(16, 16)] = row
        return carry

    lax.fori_loop(0, _RPS2 // 16, expand, 0)
    pltpu.sync_copy(dexp, out_hbm.at[pl.ds(c * _N2 + s * _RPS2, _RPS2)])


@functools.partial(
    pl.kernel,
    mesh=_sc_mesh,
    out_type=jax.ShapeDtypeStruct((2 * _N2, _H), _f32),
    scratch_types=[
        pltpu.VMEM((_NCHUNK, _K), jnp.int32),
        pltpu.VMEM((_NCHUNK, _K), jnp.int32),
        pltpu.VMEM((2, _G * _K, _H), _f32),
        pltpu.VMEM_SHARED((_N2, _H), _f32),
        pltpu.SemaphoreType.DMA,
        pltpu.SemaphoreType.DMA,
    ],
    compiler_params=_sc_params,
)
def _sc_scatter(g_hbm, src_hbm, dst_hbm, zeros2_hbm, out_hbm,
                sidx, didx, rows, acc, sem0, sem1):
    c = lax.axis_index("c")
    s = lax.axis_index("s")
    w = s * _NC + c

    pltpu.sync_copy(src_hbm.at[w], sidx)
    pltpu.sync_copy(dst_hbm.at[w], didx)
    pltpu.sync_copy(zeros2_hbm.at[pl.ds(s * _RPS2, _RPS2)],
                    acc.at[pl.ds(s * _RPS2, _RPS2)])
    plsc.subcore_barrier()

    def fire(grp, slot, sem):
        for j in range(_G):
            pltpu.async_copy(g_hbm.at[sidx.at[grp * _G + j]],
                             rows.at[slot, pl.ds(j * _K, _K)], sem)

    def drain_scatter(grp, slot, sem):
        pltpu.make_async_copy(zeros2_hbm.at[pl.ds(0, _G * _K)],
                              rows.at[slot], sem).wait()
        for j in range(_G):
            pltpu.sync_copy(rows.at[slot, pl.ds(j * _K, _K)],
                            acc.at[didx.at[grp * _G + j]], add=True)

    fire(0, 0, sem0)
    fire(1, 1, sem1)

    def body(k, carry):
        g0 = 2 * k
        drain_scatter(g0, 0, sem0)
        fire(g0 + 2, 0, sem0)
        drain_scatter(g0 + 1, 1, sem1)
        fire(g0 + 3, 1, sem1)
        return carry

    lax.fori_loop(0, (_NG - 3) // 2, body, 0)
    drain_scatter(_NG - 3, 0, sem0)
    fire(_NG - 1, 0, sem0)
    drain_scatter(_NG - 2, 1, sem1)
    drain_scatter(_NG - 1, 0, sem0)
    plsc.subcore_barrier()
    pltpu.sync_copy(acc.at[pl.ds(s * _RPS2, _RPS2)],
                    out_hbm.at[pl.ds(c * _N2 + s * _RPS2, _RPS2)])


def _tc_h1_body(x0p_ref, w1s_ref, h1_ref):
    h1_ref[...] = jnp.dot(x0p_ref[...], w1s_ref[...],
                          preferred_element_type=_f32)


def _tc_g1_body(h1_ref, degp_ref, g1_ref, dinv_ref):
    dinv = lax.rsqrt(1.0 + degp_ref[0] + degp_ref[1])
    g1_ref[...] = h1_ref[...] * dinv
    dinv_ref[...] = dinv


def _tc_mid_body(p_ref, g_ref, dinv_ref, b_ref, w_ref, gn_ref):
    dinv = dinv_ref[...]
    x = jnp.maximum(dinv * (p_ref[0] + p_ref[1] + g_ref[...]) + b_ref[...], 0.0)
    gn_ref[...] = jnp.dot(x, w_ref[...], preferred_element_type=_f32) * dinv


def _tc_head_body(p_ref, g_ref, dinv_ref, b3_ref, wl1_ref, bl1_ref,
                  wl2_ref, bl2_ref, out_ref):
    dinv = dinv_ref[...]
    x3 = jnp.maximum(dinv * (p_ref[0] + p_ref[1] + g_ref[...]) + b3_ref[...], 0.0)
    h = jnp.maximum(
        jnp.dot(x3, wl1_ref[...], preferred_element_type=_f32) + bl1_ref[...], 0.0)
    out_ref[...] = (
        jnp.dot(h, wl2_ref[...], preferred_element_type=_f32) + bl2_ref[...])


def _block_diag4(w):
    a, b = w.shape
    eye = jnp.eye(4, dtype=w.dtype)
    return (eye[:, None, :, None] * w[None, :, None, :]).reshape(4 * a, 4 * b)


def kernel(x0, edge_index, batch, W1, b1, W2, b2, W3, b3, Wl1, bl1, Wl2, bl2):
    src = edge_index[0].reshape(_NW, _NCHUNK, _K)
    dst = edge_index[1].reshape(_NW, _NCHUNK, _K)
    zeros1 = jnp.zeros((_N2,), _f32)
    zeros2 = jnp.zeros((_N2, _H), _f32)

    x0p = jnp.pad(x0, ((0, _N2 - _N), (0, 0))).reshape(_NP, 4 * _D)
    w1s = _block_diag4(W1)
    w2s = _block_diag4(W2)
    w3s = _block_diag4(W3)
    wl1s = _block_diag4(Wl1)
    wl2s = _block_diag4(Wl2)
    b1p = jnp.tile(b1, 4).reshape(1, 4 * _H)
    b2p = jnp.tile(b2, 4).reshape(1, 4 * _H)
    b3p = jnp.tile(b3, 4).reshape(1, 4 * _H)
    bl1p = jnp.tile(bl1, 4).reshape(1, 64)
    bl2p = jnp.tile(bl2, 4).reshape(1, 4 * _C)

    h1 = pl.pallas_call(
        _tc_h1_body,
        out_shape=jax.ShapeDtypeStruct((_NP, 4 * _H), _f32),
    )(x0p, w1s)

    degp = _sc_deg(dst, zeros1).reshape(2, _NP, 4 * _H)

    g1, dinvp = pl.pallas_call(
        _tc_g1_body,
        out_shape=[jax.ShapeDtypeStruct((_NP, 4 * _H), _f32),
                   jax.ShapeDtypeStruct((_NP, 4 * _H), _f32)],
    )(h1, degp)

    p1 = _sc_scatter(g1.reshape(_N2, _H), src, dst,
                     zeros2).reshape(2, _NP, 4 * _H)
    g2 = pl.pallas_call(
        _tc_mid_body,
        out_shape=jax.ShapeDtypeStruct((_NP, 4 * _H), _f32),
    )(p1, g1, dinvp, b1p, w2s)

    p2 = _sc_scatter(g2.reshape(_N2, _H), src, dst,
                     zeros2).reshape(2, _NP, 4 * _H)
    g3 = pl.pallas_call(
        _tc_mid_body,
        out_shape=jax.ShapeDtypeStruct((_NP, 4 * _H), _f32),
    )(p2, g2, dinvp, b2p, w3s)

    p3 = _sc_scatter(g3.reshape(_N2, _H), src, dst,
                     zeros2).reshape(2, _NP, 4 * _H)
    outp = pl.pallas_call(
        _tc_head_body,
        out_shape=jax.ShapeDtypeStruct((_NP, 4 * _C), _f32),
    )(p3, g3, dinvp, b3p, wl1s, bl1p, wl2s, bl2p)

    return outp.reshape(_N2, _C)[: _N]
